# Initial kernel scaffold; baseline (speedup 1.0000x reference)
import jax
import jax.numpy as jnp
from jax.experimental import pallas as pl


def kernel(x, edge_index, W1l, b1, W1r, W2l, b2, W2r, Wlin, blin):
    raise NotImplementedError("write your pallas kernel here")


# speedup vs baseline: 20.2634x; 20.2634x over previous
import functools

import jax
import jax.numpy as jnp
from jax import lax
from jax.experimental import pallas as pl
from jax.experimental.pallas import tpu as pltpu
from jax.experimental.pallas import tpu_sc as plsc

_NC = 2
_NS = 16
_C = 128
_NP = 102400


def _sc_segment_sum(zeros, tables, src, dst, with_count):
    nf = len(tables)
    e = src.shape[0]
    nw = _NC * _NS
    ew = e // nw
    n_full = ew // _C
    tail = ew - n_full * _C
    nout = nf + (1 if with_count else 0)

    mesh = plsc.VectorSubcoreMesh(core_axis_name="c", subcore_axis_name="s")

    scratch = (
        [pltpu.VMEM_SHARED((_NP,), jnp.float32) for _ in range(nf)]
        + [pltpu.VMEM_SHARED((_NP,), jnp.float32) for _ in range(nout)]
        + [
            pltpu.VMEM((1, _C), jnp.int32),
            pltpu.VMEM((1, _C), jnp.int32),
            pltpu.VMEM((nf, _C), jnp.float32),
            pltpu.VMEM((_C,), jnp.float32),
        ]
    )
    if tail:
        scratch += [
            pltpu.VMEM((tail,), jnp.int32),
            pltpu.VMEM((tail,), jnp.int32),
            pltpu.VMEM((nf, tail), jnp.float32),
            pltpu.VMEM((tail,), jnp.float32),
        ]
    scratch.append(pltpu.SemaphoreType.DMA)

    @functools.partial(
        pl.kernel,
        out_type=[jax.ShapeDtypeStruct((_NC * _NP,), jnp.float32)] * nout,
        mesh=mesh,
        scratch_types=scratch,
    )
    def body(*refs):
        zeros_hbm = refs[0]
        tab_hbm = refs[1:1 + nf]
        src_hbm = refs[1 + nf]
        dst_hbm = refs[2 + nf]
        outs = refs[3 + nf:3 + nf + nout]
        r = refs[3 + nf + nout:]
        tab_sp = r[:nf]
        acc_sp = r[nf:nf + nout]
        r = r[nf + nout:]
        if tail:
            (sbuf, dbuf, gbuf, ones_b, stail, dtail, gtail, ones_t, sem) = r
        else:
            (sbuf, dbuf, gbuf, ones_b, sem) = r

        cid = lax.axis_index("c")
        sid = lax.axis_index("s")
        wid = sid * _NC + cid
        base = wid * ew

        @pl.when(sid == 0)
        def _stage():
            for k in range(nf):
                pltpu.sync_copy(tab_hbm[k], tab_sp[k])
            for k in range(nout):
                pltpu.sync_copy(zeros_hbm, acc_sp[k])

        if with_count:
            one = jnp.ones((16,), jnp.float32)
            for i in range(_C // 16):
                ones_b[pl.ds(i * 16, 16)] = one
            if tail:
                for i in range(tail // 16):
                    ones_t[pl.ds(i * 16, 16)] = one

        plsc.subcore_barrier()

        def do_edges(off, cnt, sb, db, gb, ob):
            pltpu.sync_copy(src_hbm.at[pl.ds(off, cnt)], sb)
            pltpu.sync_copy(dst_hbm.at[pl.ds(off, cnt)], db)
            gathers = [
                pltpu.async_copy(tab_sp[k].at[sb], gb.at[k], sem)
                for k in range(nf)
            ]
            for g in gathers:
                g.wait()
            scats = [
                pltpu.async_copy(gb.at[k], acc_sp[k].at[db], sem, add=True)
                for k in range(nf)
            ]
            if with_count:
                scats.append(
                    pltpu.async_copy(ob, acc_sp[nf].at[db], sem, add=True))
            for s in scats:
                s.wait()

        def chunk(g, carry):
            do_edges(base + g * _C, _C, sbuf.at[0], dbuf.at[0], gbuf, ones_b)
            return carry

        lax.fori_loop(0, n_full, chunk, 0)

        if tail:
            do_edges(base + n_full * _C, tail, stail, dtail, gtail, ones_t)

        plsc.subcore_barrier()

        @pl.when(sid == 0)
        def _writeout():
            for k in range(nout):
                pltpu.sync_copy(acc_sp[k], outs[k].at[pl.ds(cid * _NP, _NP)])

    return body(zeros, *tables, src, dst)


def _dense_tc(a0, a1, a2, acnt, x0, x1, x2,
              w1l, b1c, w1r, w2l, w2r, wlin, b2c, blinc):
    bk = 10240
    grid = (_NP // bk,)
    nb = _NP // bk

    def body(a0r0, a0r1, a1r0, a1r1, a2r0, a2r1, cr0, cr1, x0r, x1r, x2r,
             w1l_r, b1_r, w1r_r, w2l_r, w2r_r, wlin_r, b2_r, blin_r,
             p0_r, p1_r, qc0_r, qc1_r):
        cnt = cr0[...] + cr1[...]
        inv = 1.0 / jnp.maximum(cnt, 1.0)
        m0 = (a0r0[...] + a0r1[...]) * inv
        m1 = (a1r0[...] + a1r1[...]) * inv
        m2 = (a2r0[...] + a2r1[...]) * inv
        mean = jnp.concatenate(
            [m0.reshape(1, bk), m1.reshape(1, bk), m2.reshape(1, bk)], axis=0)
        xb = jnp.concatenate(
            [x0r[...].reshape(1, bk), x1r[...].reshape(1, bk),
             x2r[...].reshape(1, bk)], axis=0)
        h = (jnp.dot(w1l_r[...], mean, preferred_element_type=jnp.float32)
             + jnp.dot(w1r_r[...], xb, preferred_element_type=jnp.float32)
             + b1_r[...])
        h = jnp.maximum(h, 0.0)
        amat = jnp.dot(wlin_r[...], w2l_r[...],
                       preferred_element_type=jnp.float32)
        bmat = jnp.dot(wlin_r[...], w2r_r[...],
                       preferred_element_type=jnp.float32)
        cvec = (jnp.dot(wlin_r[...], b2_r[...],
                        preferred_element_type=jnp.float32)
                + blin_r[...])
        p = jnp.dot(amat, h, preferred_element_type=jnp.float32)
        q = jnp.dot(bmat, h, preferred_element_type=jnp.float32) + cvec
        p0_r[...] = p[0:1, :].reshape(bk)
        p1_r[...] = p[1:2, :].reshape(bk)
        qc0_r[...] = q[0:1, :].reshape(bk)
        qc1_r[...] = q[1:2, :].reshape(bk)

    part = lambda: pl.BlockSpec((bk,), lambda i: (i,))
    part1 = lambda: pl.BlockSpec((bk,), lambda i: (nb + i,))
    wfull = lambda a, b: pl.BlockSpec((a, b), lambda i: (0, 0))
    return pl.pallas_call(
        body,
        grid=grid,
        in_specs=[
            part(), part1(), part(), part1(), part(), part1(), part(), part1(),
            part(), part(), part(),
            wfull(32, 3), wfull(32, 1), wfull(32, 3),
            wfull(16, 32), wfull(16, 32), wfull(2, 16),
            wfull(16, 1), wfull(2, 1),
        ],
        out_specs=[part(), part(), part(), part()],
        out_shape=[jax.ShapeDtypeStruct((_NP,), jnp.float32)] * 4,
    )(a0, a0, a1, a1, a2, a2, acnt, acnt, x0, x1, x2,
      w1l, b1c, w1r, w2l, w2r, wlin, b2c, blinc)


def _combine_tc(c0, c1, acnt, qc0, qc1):
    bk = 10240
    nb = _NP // bk

    def body(c0r0, c0r1, c1r0, c1r1, cr0, cr1, qc0_r, qc1_r, o0_r, o1_r):
        inv = 1.0 / jnp.maximum(cr0[...] + cr1[...], 1.0)
        o0_r[...] = (c0r0[...] + c0r1[...]) * inv + qc0_r[...]
        o1_r[...] = (c1r0[...] + c1r1[...]) * inv + qc1_r[...]

    part = lambda: pl.BlockSpec((bk,), lambda i: (i,))
    part1 = lambda: pl.BlockSpec((bk,), lambda i: (nb + i,))
    return pl.pallas_call(
        body,
        grid=(_NP // bk,),
        in_specs=[part(), part1(), part(), part1(), part(), part1(),
                  part(), part()],
        out_specs=[part(), part()],
        out_shape=[jax.ShapeDtypeStruct((_NP,), jnp.float32)] * 2,
    )(c0, c0, c1, c1, acnt, acnt, qc0, qc1)


def kernel(x, edge_index, W1l, b1, W1r, W2l, b2, W2r, Wlin, blin):
    n = x.shape[0]
    src = edge_index[0]
    dst = edge_index[1]
    pad = jnp.zeros((_NP - n,), jnp.float32)
    x0 = jnp.concatenate([x[:, 0], pad])
    x1 = jnp.concatenate([x[:, 1], pad])
    x2 = jnp.concatenate([x[:, 2], pad])
    zeros = jnp.zeros((_NP,), jnp.float32)

    a0, a1, a2, acnt = _sc_segment_sum(
        zeros, [x0, x1, x2], src, dst, with_count=True)

    p0, p1, qc0, qc1 = _dense_tc(
        a0, a1, a2, acnt, x0, x1, x2,
        W1l, b1.reshape(-1, 1), W1r, W2l, W2r, Wlin,
        b2.reshape(-1, 1), blin.reshape(-1, 1))

    c0, c1 = _sc_segment_sum(zeros, [p0, p1], src, dst, with_count=False)
    o0, o1 = _combine_tc(c0, c1, acnt, qc0, qc1)
    return jnp.stack([o0[:n], o1[:n]], axis=1)
